# Initial kernel scaffold; baseline (speedup 1.0000x reference)
import jax
import jax.numpy as jnp
from jax.experimental import pallas as pl


def kernel(node_feats, coords, batch_index, W1, W2):
    raise NotImplementedError("write your pallas kernel here")


# speedup vs baseline: 2.1716x; 2.1716x over previous
import functools

import jax
import jax.numpy as jnp
import numpy as np
from jax.experimental import pallas as pl
from jax.experimental.pallas import tpu as pltpu

RADIUS = 0.25
D_MIN = 0.0
D_MAX = 0.25
NB = 32
EPS = 1e-12

_CENTERS = np.linspace(D_MIN, D_MAX, NB).astype(np.float32)
_GAMMA = np.float32(1.0) / (_CENTERS[1] - _CENTERS[0]) ** 2

BI = 8
BJ = 128
P = BI * BJ


def _cfconv_kernel(ilo_ref, nib_ref, c_ref, f_ref, si_ref, sj_ref,
                   w1_ref, w2_ref, cen_ref, out_ref):
    jb = pl.program_id(0)
    i0 = ilo_ref[jb]
    nib = nib_ref[jb]

    si = si_ref[...]
    sj = sj_ref[...]
    w1 = w1_ref[...]
    w2 = w2_ref[...]

    cj = c_ref[pl.ds(jb * BJ, BJ), :]
    pj = jnp.dot(sj, cj, precision=jax.lax.Precision.HIGHEST,
                 preferred_element_type=jnp.float32)

    centers = cen_ref[...]
    gamma = float(_GAMMA)

    iota_p = jax.lax.broadcasted_iota(jnp.int32, (P, 1), 0)
    i_loc = iota_p // BJ
    j_loc = iota_p % BJ
    jg = jb * BJ + j_loc

    def body(t, acc):
        ib = i0 + t * BI
        ci = c_ref[pl.ds(ib, BI), :]
        pi = jnp.dot(si, ci, precision=jax.lax.Precision.HIGHEST,
                     preferred_element_type=jnp.float32)
        fi = f_ref[pl.ds(ib, BI), :]

        dx = pi[:, 0:1] - pj[:, 0:1]
        dy = pi[:, 1:2] - pj[:, 1:2]
        dz = pi[:, 2:3] - pj[:, 2:3]
        d2_diff = dx * dx + dy * dy + dz * dz
        bf = lambda v: v.astype(jnp.bfloat16).astype(jnp.float32)
        d2_dot = (pi[:, 3:4] + pj[:, 3:4]
                  - 2.0 * (bf(pi[:, 0:1]) * bf(pj[:, 0:1])
                           + bf(pi[:, 1:2]) * bf(pj[:, 1:2])
                           + bf(pi[:, 2:3]) * bf(pj[:, 2:3])))
        same_batch = pi[:, 4:5] == pj[:, 4:5]
        ig = ib + i_loc
        mask = same_batch & (ig != jg) & (d2_dot <= RADIUS * RADIUS)

        d = jnp.sqrt(d2_diff + EPS)
        rbf = jnp.exp(-gamma * (d - centers) ** 2)
        h1 = jnp.maximum(
            jnp.dot(rbf, w1, preferred_element_type=jnp.float32), 0.0)
        m = jnp.maximum(
            jnp.dot(h1, w2, preferred_element_type=jnp.float32), 0.0)
        m = m * mask.astype(jnp.float32)

        m3 = m.reshape(BI, BJ, -1)
        contrib = jnp.sum(m3 * fi[:, None, :], axis=0)
        return acc + contrib

    hid = out_ref.shape[-1]
    acc = jax.lax.fori_loop(0, nib, body,
                            jnp.zeros((BJ, hid), jnp.float32))
    out_ref[...] = acc


@functools.partial(jax.jit, static_argnames=())
def kernel(node_feats, coords, batch_index, W1, W2):
    V, HID = node_feats.shape
    NJ = (V + BJ - 1) // BJ
    VP = NJ * BJ

    starts = jnp.minimum(jnp.arange(NJ, dtype=jnp.int32) * BJ, V - 1)
    ends = jnp.minimum(starts + (BJ - 1), V - 1)
    b_lo = batch_index[starts]
    b_hi = batch_index[ends]
    ilo = jnp.searchsorted(batch_index, b_lo, side="left").astype(jnp.int32)
    ihi = jnp.searchsorted(batch_index, b_hi, side="right").astype(jnp.int32)
    ilo_al = (ilo // BI) * BI
    nib = (ihi - ilo_al + BI - 1) // BI

    sq = jnp.sum(coords * coords, axis=-1)
    c_tab = jnp.zeros((VP, 8), jnp.float32)
    c_tab = c_tab.at[:V, 0:3].set(coords)
    c_tab = c_tab.at[:V, 3].set(sq)
    c_tab = c_tab.at[:V, 4].set(batch_index.astype(jnp.float32))
    c_tab = c_tab.at[V:, 4].set(-1.0)

    f_tab = jnp.zeros((VP, HID), node_feats.dtype).at[:V].set(node_feats)

    si = jnp.kron(jnp.eye(BI, dtype=jnp.float32),
                  jnp.ones((BJ, 1), jnp.float32))
    sj = jnp.kron(jnp.ones((BI, 1), jnp.float32),
                  jnp.eye(BJ, dtype=jnp.float32))

    full = lambda shape: pl.BlockSpec(shape, lambda j: (0, 0))
    out = pl.pallas_call(
        _cfconv_kernel,
        grid=(NJ,),
        in_specs=[
            pl.BlockSpec(memory_space=pltpu.SMEM),
            pl.BlockSpec(memory_space=pltpu.SMEM),
            full((VP, 8)),
            full((VP, HID)),
            full((P, BI)),
            full((P, BJ)),
            full((NB, HID)),
            full((HID, HID)),
            full((1, NB)),
        ],
        out_specs=pl.BlockSpec((BJ, HID), lambda j: (j, 0)),
        out_shape=jax.ShapeDtypeStruct((VP, HID), jnp.float32),
    )(ilo_al, nib, c_tab, f_tab, si, sj, W1, W2,
      jnp.asarray(_CENTERS).reshape(1, NB))
    return out[:V]
